# Initial kernel scaffold; baseline (speedup 1.0000x reference)
import jax
import jax.numpy as jnp
from jax.experimental import pallas as pl


def kernel(vertices, edge_index, W1, b1, W2, b2, W3, b3, Wm1, bm1, Wm2, bm2):
    raise NotImplementedError("write your pallas kernel here")


# speedup vs baseline: 7.5818x; 7.5818x over previous
import functools

import jax
import jax.numpy as jnp
from jax import lax
from jax.experimental import pallas as pl
from jax.experimental.pallas import tpu as pltpu
from jax.experimental.pallas import tpu_sc as plsc

N = 1024
E = 16384
D = 16
LANES = 16
MAX_DEG = 512

def _half_bounds(base, half):
    bs = []
    p = base + 864
    for _ in range(3):
        bs.append(p)
        p += 864
    p = base + 3 * 864 + 432
    while p < base + half:
        bs.append(p)
        p += 432
    return bs


_BOUNDS = _half_bounds(0, E // 2) + [E // 2] + _half_bounds(E // 2, E // 2)


@functools.cache
def _make_densify():
    num_cores, num_subcores = 2, 16
    nw = num_cores * num_subcores
    rows_per_w = N // nw
    words_per_w = rows_per_w * N
    mesh = plsc.VectorSubcoreMesh(core_axis_name="c", subcore_axis_name="s")

    @functools.partial(
        pl.kernel,
        mesh=mesh,
        out_type=jax.ShapeDtypeStruct((N * N,), jnp.float32),
        scratch_types=[
            pltpu.VMEM((E,), jnp.int32),
            pltpu.VMEM((E,), jnp.int32),
            pltpu.VMEM((words_per_w,), jnp.float32),
        ],
        compiler_params=pltpu.CompilerParams(needs_layout_passes=False),
    )
    def densify(edge_hbm, a0_hbm, src_v, dst_v, acc_v):
        wid = lax.axis_index("c") * num_subcores + lax.axis_index("s")
        base_row = wid * rows_per_w

        zeros16 = jnp.zeros((LANES,), jnp.float32)
        ones16 = jnp.ones((LANES,), jnp.float32)

        def zero_body(i, carry):
            acc_v[pl.ds(i * LANES, LANES)] = zeros16
            return carry

        lax.fori_loop(0, words_per_w // LANES, zero_body, 0)

        pltpu.sync_copy(edge_hbm.at[0], src_v)
        pltpu.sync_copy(edge_hbm.at[1], dst_v)

        def edge_body(i, carry):
            s = src_v[pl.ds(i * LANES, LANES)]
            d = dst_v[pl.ds(i * LANES, LANES)]
            local = d - base_row
            mask = (local >= 0) & (local < rows_per_w)
            idx = jnp.where(mask, local * N + s, 0)
            plsc.addupdate_scatter(acc_v, [idx], ones16, mask=mask)
            return carry

        lax.fori_loop(0, E // LANES, edge_body, 0)

        pltpu.sync_copy(acc_v, a0_hbm.at[pl.ds(wid * words_per_w, words_per_w)])

    return densify


def _tc_scan_body(vert_ref, dis2r_ref, srcT_ref, nrmT_ref, newT_ref,
                  maxk_ref, w1_ref, b1_ref, w2_ref, b2_ref, w3_ref, b3_ref,
                  wm1_ref, bm1_ref, wm2_ref, bm2_ref, out_ref, v_ref):
    dis2r = dis2r_ref[...]
    maxk = maxk_ref[0, 0]

    v_ref[...] = vert_ref[...]

    w1 = w1_ref[...]
    b1 = b1_ref[...]
    w2 = w2_ref[...]
    b2 = b2_ref[...]
    w3 = w3_ref[...]
    b3 = b3_ref[...]
    wm1 = wm1_ref[...]
    bm1 = bm1_ref[...]
    wm2 = wm2_ref[...]
    bm2 = bm2_ref[...]
    row_iota = lax.broadcasted_iota(jnp.int32, (N, 1), 0)

    def agg_exact(h):
        hT = h.T
        zero = hT * 0.0
        self_init = hT * dis2r

        def slot_body(j, carry):
            acc, part = carry
            idx = srcT_ref[pl.ds(j, 1), :]
            nrm = nrmT_ref[pl.ds(j, 1), :]
            nw = newT_ref[pl.ds(j, 1), :] > 0.5
            g = zero
            for c in range(N // 128):
                lidx = idx - (c * 128)
                inb = (lidx >= 0) & (lidx < 128)
                cl = jnp.clip(lidx, 0, 127)
                clb = jnp.broadcast_to(cl, (D, N))
                tbl = lax.slice(hT, (0, c * 128), (D, (c + 1) * 128))
                got = jnp.take_along_axis(tbl, clb, axis=1)
                g = g + got * inb.astype(jnp.float32)
            g = g * nrm
            acc2 = jnp.where(nw, acc + part, acc)
            part2 = jnp.where(nw, g, part + g)
            return acc2, part2

        acc, part = lax.fori_loop(0, maxk, slot_body, (self_init, zero))
        return (acc + part).T

    def layer(v, w, b):
        h = jnp.dot(v, w, preferred_element_type=jnp.float32) + b
        return jnp.tanh(agg_exact(h))

    def body(t, carry):
        v = v_ref[...]
        v = layer(v, w1, b1)
        v = layer(v, w2, b2)
        v = layer(v, w3, b3)
        v_ref[...] = v
        p1 = jnp.maximum(jnp.dot(v, wm1, preferred_element_type=jnp.float32) + bm1, 0.0)
        p = jnp.dot(p1, wm2, preferred_element_type=jnp.float32) + bm2
        m = jnp.max(p)
        idx = jnp.min(jnp.where(p >= m, row_iota, N))
        out_ref[pl.ds(t, 1), :] = v_ref[pl.ds(idx, 1), :]
        return carry

    lax.fori_loop(0, N, body, 0)


def kernel(vertices, edge_index, W1, b1, W2, b2, W3, b3, Wm1, bm1, Wm2, bm2):
    src = edge_index[0]
    dst = edge_index[1]

    a0 = _make_densify()(edge_index).reshape(N, N)
    deg = jnp.sum(a0, axis=1) + 1.0
    dis = lax.rsqrt(deg)

    order = jnp.argsort(dst, stable=True)
    src_s = src[order]
    dst_s = dst[order]
    norm_s = dis[src_s] * dis[dst_s]

    counts = jnp.sum(a0, axis=1).astype(jnp.int32)
    starts = jnp.cumsum(counts) - counts
    pos = jnp.arange(E, dtype=jnp.int32)
    slot = pos - starts[dst_s]

    bounds = jnp.asarray(_BOUNDS, dtype=jnp.int32)
    newflag = (jnp.isin(pos, bounds) & (slot > 0)).astype(jnp.float32)

    srcT = jnp.zeros((MAX_DEG, N), jnp.int32).at[slot, dst_s].set(src_s, mode="drop")
    nrmT = jnp.zeros((MAX_DEG, N), jnp.float32).at[slot, dst_s].set(norm_s, mode="drop")
    newT = jnp.zeros((MAX_DEG, N), jnp.float32).at[slot, dst_s].set(newflag, mode="drop")
    maxk = jnp.max(counts).reshape(1, 1)

    out = pl.pallas_call(
        _tc_scan_body,
        out_shape=jax.ShapeDtypeStruct((N, D), jnp.float32),
        scratch_shapes=[pltpu.VMEM((N, D), jnp.float32)],
    )(vertices, (dis * dis).reshape(1, N),
      srcT, nrmT, newT, maxk,
      W1, b1.reshape(1, D), W2, b2.reshape(1, D), W3, b3.reshape(1, D),
      Wm1, bm1.reshape(1, D), Wm2, bm2.reshape(1, 1))
    return out


# speedup vs baseline: 9.4915x; 1.2519x over previous
import functools

import jax
import jax.numpy as jnp
from jax import lax
from jax.experimental import pallas as pl
from jax.experimental.pallas import tpu as pltpu
from jax.experimental.pallas import tpu_sc as plsc

N = 1024
E = 16384
D = 16
LANES = 16
MAX_DEG = 512

def _half_bounds(base, half):
    bs = []
    p = base + 864
    for _ in range(3):
        bs.append(p)
        p += 864
    p = base + 3 * 864 + 432
    while p < base + half:
        bs.append(p)
        p += 432
    return bs


_BOUNDS = _half_bounds(0, E // 2) + [E // 2] + _half_bounds(E // 2, E // 2)


@functools.cache
def _make_densify():
    num_cores, num_subcores = 2, 16
    nw = num_cores * num_subcores
    rows_per_w = N // nw
    words_per_w = rows_per_w * N
    mesh = plsc.VectorSubcoreMesh(core_axis_name="c", subcore_axis_name="s")

    @functools.partial(
        pl.kernel,
        mesh=mesh,
        out_type=jax.ShapeDtypeStruct((N * N,), jnp.float32),
        scratch_types=[
            pltpu.VMEM((E,), jnp.int32),
            pltpu.VMEM((E,), jnp.int32),
            pltpu.VMEM((words_per_w,), jnp.float32),
        ],
        compiler_params=pltpu.CompilerParams(needs_layout_passes=False),
    )
    def densify(edge_hbm, a0_hbm, src_v, dst_v, acc_v):
        wid = lax.axis_index("c") * num_subcores + lax.axis_index("s")
        base_row = wid * rows_per_w

        zeros16 = jnp.zeros((LANES,), jnp.float32)
        ones16 = jnp.ones((LANES,), jnp.float32)

        def zero_body(i, carry):
            acc_v[pl.ds(i * LANES, LANES)] = zeros16
            return carry

        lax.fori_loop(0, words_per_w // LANES, zero_body, 0)

        pltpu.sync_copy(edge_hbm.at[0], src_v)
        pltpu.sync_copy(edge_hbm.at[1], dst_v)

        def edge_body(i, carry):
            s = src_v[pl.ds(i * LANES, LANES)]
            d = dst_v[pl.ds(i * LANES, LANES)]
            local = d - base_row
            mask = (local >= 0) & (local < rows_per_w)
            idx = jnp.where(mask, local * N + s, 0)
            plsc.addupdate_scatter(acc_v, [idx], ones16, mask=mask)
            return carry

        lax.fori_loop(0, E // LANES, edge_body, 0)

        pltpu.sync_copy(acc_v, a0_hbm.at[pl.ds(wid * words_per_w, words_per_w)])

    return densify


def _tc_scan_body(vert_ref, dis2r_ref, srcT_ref, nrmT_ref, newT_ref,
                  maxk_ref, permcol_ref, w1_ref, b1_ref, w2_ref, b2_ref,
                  w3_ref, b3_ref, wm1_ref, bm1_ref, wm2_ref, bm2_ref,
                  out_ref, v_ref):
    dis2r = dis2r_ref[...]
    maxk = maxk_ref[0, 0]
    tnar = maxk_ref[0, 1]
    permcol = permcol_ref[...]

    v_ref[...] = vert_ref[...]

    w1 = w1_ref[...]
    b1 = b1_ref[...]
    w2 = w2_ref[...]
    b2 = b2_ref[...]
    w3 = w3_ref[...]
    b3 = b3_ref[...]
    wm1 = wm1_ref[...]
    bm1 = bm1_ref[...]
    wm2 = wm2_ref[...]
    bm2 = bm2_ref[...]
    row_iota = lax.broadcasted_iota(jnp.int32, (N, 1), 0)

    def agg_exact(h):
        hT = h.T
        zero = hT * 0.0
        self_init = hT * dis2r

        def fold(j, acc, part, width):
            idx = lax.slice(srcT_ref[pl.ds(j, 1), :], (0, 0), (1, width))
            nrm = lax.slice(nrmT_ref[pl.ds(j, 1), :], (0, 0), (1, width))
            nw = lax.slice(newT_ref[pl.ds(j, 1), :], (0, 0), (1, width)) > 0.5
            g = acc * 0.0
            for c in range(N // 128):
                lidx = idx - (c * 128)
                inb = (lidx >= 0) & (lidx < 128)
                cl = jnp.clip(lidx, 0, 127)
                clb = jnp.broadcast_to(cl, (D, width))
                tbl = lax.slice(hT, (0, c * 128), (D, (c + 1) * 128))
                got = jnp.take_along_axis(tbl, clb, axis=1)
                g = g + got * inb.astype(jnp.float32)
            g = g * nrm
            acc2 = jnp.where(nw, acc + part, acc)
            part2 = jnp.where(nw, g, part + g)
            return acc2, part2

        def wide_body(j, carry):
            acc, part = carry
            return fold(j, acc, part, N)

        def nar_body(j, carry):
            acc, part = carry
            return fold(j, acc, part, 128)

        acc, part = lax.fori_loop(0, jnp.minimum(maxk, tnar), wide_body,
                                  (self_init, zero))
        acc128, part128 = lax.fori_loop(
            tnar, maxk, nar_body,
            (lax.slice(acc, (0, 0), (D, 128)), lax.slice(part, (0, 0), (D, 128))))
        acc = jnp.concatenate([acc128, lax.slice(acc, (0, 128), (D, N))], axis=1)
        part = jnp.concatenate([part128, lax.slice(part, (0, 128), (D, N))], axis=1)
        return (acc + part).T

    def layer(v, w, b):
        h = jnp.dot(v, w, preferred_element_type=jnp.float32) + b
        return jnp.tanh(agg_exact(h))

    def body(t, carry):
        v = v_ref[...]
        v = layer(v, w1, b1)
        v = layer(v, w2, b2)
        v = layer(v, w3, b3)
        v_ref[...] = v
        p1 = jnp.maximum(jnp.dot(v, wm1, preferred_element_type=jnp.float32) + bm1, 0.0)
        p = jnp.dot(p1, wm2, preferred_element_type=jnp.float32) + bm2
        m = jnp.max(p)
        idx0 = jnp.min(jnp.where(p >= m, permcol, N))
        pos = jnp.min(jnp.where((p >= m) & (permcol == idx0), row_iota, N))
        out_ref[pl.ds(t, 1), :] = v_ref[pl.ds(pos, 1), :]
        return carry

    lax.fori_loop(0, N, body, 0)


def kernel(vertices, edge_index, W1, b1, W2, b2, W3, b3, Wm1, bm1, Wm2, bm2):
    src = edge_index[0]
    dst = edge_index[1]

    a0 = _make_densify()(edge_index).reshape(N, N)
    deg = jnp.sum(a0, axis=1) + 1.0
    dis = lax.rsqrt(deg)

    order = jnp.argsort(dst, stable=True)
    src_s = src[order]
    dst_s = dst[order]
    norm_s = dis[src_s] * dis[dst_s]

    counts = jnp.sum(a0, axis=1).astype(jnp.int32)
    starts = jnp.cumsum(counts) - counts
    pos = jnp.arange(E, dtype=jnp.int32)
    slot = pos - starts[dst_s]

    bounds = jnp.asarray(_BOUNDS, dtype=jnp.int32)
    newflag = (jnp.isin(pos, bounds) & (slot > 0)).astype(jnp.float32)

    perm = jnp.argsort(-counts, stable=True)
    posof = jnp.zeros((N,), jnp.int32).at[perm].set(
        jnp.arange(N, dtype=jnp.int32))

    srcp_s = posof[src_s]
    dstp_s = posof[dst_s]
    srcT = jnp.zeros((MAX_DEG, N), jnp.int32).at[slot, dstp_s].set(srcp_s, mode="drop")
    nrmT = jnp.zeros((MAX_DEG, N), jnp.float32).at[slot, dstp_s].set(norm_s, mode="drop")
    newT = jnp.zeros((MAX_DEG, N), jnp.float32).at[slot, dstp_s].set(newflag, mode="drop")
    tnar = jnp.sort(counts)[N - 129]
    maxk = jnp.stack([jnp.max(counts), tnar]).reshape(1, 2)

    out = pl.pallas_call(
        _tc_scan_body,
        out_shape=jax.ShapeDtypeStruct((N, D), jnp.float32),
        scratch_shapes=[pltpu.VMEM((N, D), jnp.float32)],
    )(vertices[perm], (dis * dis)[perm].reshape(1, N),
      srcT, nrmT, newT, maxk, perm.reshape(N, 1),
      W1, b1.reshape(1, D), W2, b2.reshape(1, D), W3, b3.reshape(1, D),
      Wm1, bm1.reshape(1, D), Wm2, bm2.reshape(1, 1))
    return out
